# baseline (speedup 1.0000x reference)
import functools

import jax
import jax.numpy as jnp
from jax import lax
from jax.experimental import pallas as pl
from jax.experimental.pallas import tpu as pltpu
from jax.experimental.pallas import tpu_sc as plsc

N_LATENT = 32
N_OC = 16
B = 512
R = 4096
CHUNK = 128
N_CHUNKS = R // CHUNK
LANES = 16


def _sc_gather_transposed(idx, h2):
    mesh = plsc.VectorSubcoreMesh(core_axis_name="c", subcore_axis_name="s")

    @functools.partial(
        pl.kernel,
        mesh=mesh,
        out_type=jax.ShapeDtypeStruct((N_LATENT, R, N_OC), jnp.float32),
        scratch_types=[
            pltpu.VMEM((R,), jnp.int32),
            pltpu.VMEM((R,), jnp.int32),
            pltpu.VMEM((R, N_OC), jnp.float32),
            pltpu.SemaphoreType.DMA,
        ],
        compiler_params=pltpu.CompilerParams(use_tc_tiling_on_sc=False),
    )
    def gather_kernel(idx_hbm, h2_hbm, out_hbm, idx_v, idx2_v, rows_v, sem):
        w = lax.axis_index("s") * 2 + lax.axis_index("c")
        pltpu.sync_copy(idx_hbm, idx_v)

        def scale(i, carry):
            seg = idx_v[pl.ds(i * LANES, LANES)]
            idx2_v[pl.ds(i * LANES, LANES)] = seg * N_LATENT + w
            return carry

        lax.fori_loop(0, R // LANES, scale, 0)

        def fire(j, carry):
            pltpu.make_async_copy(
                h2_hbm.at[idx2_v.at[pl.ds(j * CHUNK, CHUNK)]],
                rows_v.at[pl.ds(j * CHUNK, CHUNK)],
                sem,
            ).start()
            return carry

        lax.fori_loop(0, N_CHUNKS, fire, 0)

        def drain(j, carry):
            pltpu.make_async_copy(
                h2_hbm.at[idx2_v.at[pl.ds(j * CHUNK, CHUNK)]],
                rows_v.at[pl.ds(j * CHUNK, CHUNK)],
                sem,
            ).wait()
            return carry

        lax.fori_loop(0, N_CHUNKS, drain, 0)
        pltpu.sync_copy(rows_v, out_hbm.at[w])

    return gather_kernel(idx, h2)


def _height_matmul(latent, dhwT):
    NB = 2048

    def body(lat_ref, g_ref, out_ref):
        out_ref[...] = jnp.dot(
            lat_ref[...], g_ref[...], preferred_element_type=jnp.float32
        )

    return pl.pallas_call(
        body,
        grid=(R * N_OC // NB,),
        in_specs=[
            pl.BlockSpec((B, N_LATENT), lambda i: (0, 0)),
            pl.BlockSpec((N_LATENT, NB), lambda i: (0, i)),
        ],
        out_specs=pl.BlockSpec((B, NB), lambda i: (0, i)),
        out_shape=jax.ShapeDtypeStruct((B, R * N_OC), jnp.float32),
    )(latent, dhwT)


def _baseline_matmul(latent, baseline_w):
    NB = 2048
    n_regions = baseline_w.shape[0]

    def body(lat_ref, bw_ref, out_ref):
        out_ref[...] = lax.dot_general(
            lat_ref[...],
            bw_ref[...],
            dimension_numbers=(((1,), (1,)), ((), ())),
            preferred_element_type=jnp.float32,
        )

    return pl.pallas_call(
        body,
        grid=(pl.cdiv(n_regions, NB),),
        in_specs=[
            pl.BlockSpec((B, N_LATENT), lambda i: (0, 0)),
            pl.BlockSpec((NB, N_LATENT), lambda i: (i, 0)),
        ],
        out_specs=pl.BlockSpec((B, NB), lambda i: (0, i)),
        out_shape=jax.ShapeDtypeStruct((B, n_regions), jnp.float32),
    )(latent, baseline_w)


def kernel(latent, regions_oi, height_w, baseline_w):
    n_regions = height_w.shape[0]
    h2 = height_w.reshape(n_regions * N_LATENT, N_OC)
    dhwT = _sc_gather_transposed(regions_oi, h2).reshape(N_LATENT, R * N_OC)
    delta_height = _height_matmul(latent, dhwT).reshape(B, R, N_OC)
    delta_baseline = _baseline_matmul(latent, baseline_w)
    return delta_height, delta_baseline


# speedup vs baseline: 1.2244x; 1.2244x over previous
import functools

import jax
import jax.numpy as jnp
from jax import lax
from jax.experimental import pallas as pl
from jax.experimental.pallas import tpu as pltpu
from jax.experimental.pallas import tpu_sc as plsc

N_LATENT = 32
N_OC = 16
B = 512
R = 4096
RH = R // 2
CHUNK = 128
LANES = 16


def _sc_gather_transposed(idx, h2):
    mesh = plsc.VectorSubcoreMesh(core_axis_name="c", subcore_axis_name="s")

    @functools.partial(
        pl.kernel,
        mesh=mesh,
        out_type=jax.ShapeDtypeStruct((N_LATENT, N_OC, R), jnp.float32),
        scratch_types=[
            pltpu.VMEM((R,), jnp.int32),
            pltpu.VMEM((RH, N_OC), jnp.float32),
            pltpu.VMEM((N_OC, RH), jnp.float32),
            pltpu.SemaphoreType.DMA,
        ],
        compiler_params=pltpu.CompilerParams(
            use_tc_tiling_on_sc=False, needs_layout_passes=False
        ),
    )
    def gather_kernel(idx_hbm, h2_hbm, out_hbm, idx_v, rows_v, trans_v, sem):
        w = lax.axis_index("s") * 2 + lax.axis_index("c")
        pltpu.sync_copy(idx_hbm, idx_v)

        def scale(i, carry):
            seg = idx_v[pl.ds(i * LANES, LANES)]
            idx_v[pl.ds(i * LANES, LANES)] = seg * N_LATENT + w
            return carry

        lax.fori_loop(0, R // LANES, scale, 0)

        o_iota = lax.iota(jnp.int32, LANES)

        def do_half(h, carry):
            base = h * RH

            def fire(j, c):
                pltpu.make_async_copy(
                    h2_hbm.at[idx_v.at[pl.ds(base + j * CHUNK, CHUNK)]],
                    rows_v.at[pl.ds(j * CHUNK, CHUNK)],
                    sem,
                ).start()
                return c

            lax.fori_loop(0, RH // CHUNK, fire, 0)

            def drain(j, c):
                pltpu.make_async_copy(
                    h2_hbm.at[idx_v.at[pl.ds(base + j * CHUNK, CHUNK)]],
                    rows_v.at[pl.ds(j * CHUNK, CHUNK)],
                    sem,
                ).wait()
                return c

            lax.fori_loop(0, RH // CHUNK, drain, 0)

            def transpose(r, c):
                val = rows_v[r, :]
                plsc.store_scatter(
                    trans_v, [o_iota, jnp.full((LANES,), r, jnp.int32)], val
                )
                return c

            lax.fori_loop(0, RH, transpose, 0)
            pltpu.sync_copy(trans_v, out_hbm.at[w, :, pl.ds(base, RH)])
            return carry

        lax.fori_loop(0, 2, do_half, 0)

    return gather_kernel(idx, h2)


def _height_matmul(latent, g2):
    NB = 2048

    def body(lat_ref, g_ref, out_ref):
        out_ref[...] = jnp.dot(
            lat_ref[...], g_ref[...], preferred_element_type=jnp.float32
        )

    return pl.pallas_call(
        body,
        grid=(N_OC * R // NB,),
        in_specs=[
            pl.BlockSpec((B, N_LATENT), lambda i: (0, 0)),
            pl.BlockSpec((N_LATENT, NB), lambda i: (0, i)),
        ],
        out_specs=pl.BlockSpec((B, NB), lambda i: (0, i)),
        out_shape=jax.ShapeDtypeStruct((B, N_OC * R), jnp.float32),
    )(latent, g2)


def _baseline_matmul_t(bwT, latent):
    NB = 2048
    n_regions = bwT.shape[1]

    def body(bw_ref, lat_ref, out_ref):
        out_ref[...] = lax.dot_general(
            bw_ref[...],
            lat_ref[...],
            dimension_numbers=(((0,), (1,)), ((), ())),
            preferred_element_type=jnp.float32,
        )

    return pl.pallas_call(
        body,
        grid=(pl.cdiv(n_regions, NB),),
        in_specs=[
            pl.BlockSpec((N_LATENT, NB), lambda i: (0, i)),
            pl.BlockSpec((B, N_LATENT), lambda i: (0, 0)),
        ],
        out_specs=pl.BlockSpec((NB, B), lambda i: (i, 0)),
        out_shape=jax.ShapeDtypeStruct((n_regions, B), jnp.float32),
    )(bwT, latent)


def kernel(latent, regions_oi, height_w, baseline_w):
    n_regions = height_w.shape[0]
    h2 = height_w.reshape(n_regions * N_LATENT, N_OC)
    g = _sc_gather_transposed(regions_oi, h2)
    g2 = g.reshape(N_LATENT, N_OC * R)
    dh_p = _height_matmul(latent, g2)
    delta_height = jnp.transpose(dh_p.reshape(B, N_OC, R), (0, 2, 1))
    db_t = _baseline_matmul_t(baseline_w.T, latent)
    delta_baseline = db_t.T
    return delta_height, delta_baseline


# speedup vs baseline: 1.3493x; 1.1020x over previous
import functools

import jax
import jax.numpy as jnp
from jax import lax
from jax.experimental import pallas as pl
from jax.experimental.pallas import tpu as pltpu
from jax.experimental.pallas import tpu_sc as plsc

N_LATENT = 32
N_OC = 16
B = 512
R = 4096
LANES = 16
N_WORKERS = 32
RPW = R // N_WORKERS
HALF = RPW // 2


def _sc_gather_transposed(idx, height_w):
    mesh = plsc.VectorSubcoreMesh(core_axis_name="c", subcore_axis_name="s")

    @functools.partial(
        pl.kernel,
        mesh=mesh,
        out_type=jax.ShapeDtypeStruct((N_LATENT, N_OC, R), jnp.float32),
        scratch_types=[
            pltpu.VMEM((R,), jnp.int32),
            pltpu.VMEM((HALF, N_LATENT, N_OC), jnp.float32),
            pltpu.VMEM((N_LATENT, N_OC, HALF), jnp.float32),
            pltpu.SemaphoreType.DMA,
        ],
        compiler_params=pltpu.CompilerParams(
            use_tc_tiling_on_sc=False, needs_layout_passes=False
        ),
    )
    def gather_kernel(idx_hbm, h_hbm, out_hbm, idx_v, rows_v, trans_v, sem):
        w = lax.axis_index("s") * 2 + lax.axis_index("c")
        pltpu.sync_copy(idx_hbm, idx_v)
        o_iota = lax.iota(jnp.int32, LANES)

        def do_half(h, carry):
            base_r = w * RPW + h * HALF
            pltpu.make_async_copy(
                h_hbm.at[idx_v.at[pl.ds(base_r, HALF)]], rows_v, sem
            ).start()
            pltpu.make_async_copy(
                h_hbm.at[idx_v.at[pl.ds(base_r, HALF)]], rows_v, sem
            ).wait()

            def extract_r(r, c):
                def extract_k(k, c2):
                    val = rows_v[r, k, :]
                    plsc.store_scatter(
                        trans_v,
                        [
                            jnp.full((LANES,), k, jnp.int32),
                            o_iota,
                            jnp.full((LANES,), r, jnp.int32),
                        ],
                        val,
                    )
                    return c2

                return lax.fori_loop(0, N_LATENT, extract_k, c)

            lax.fori_loop(0, HALF, extract_r, 0)
            pltpu.sync_copy(trans_v, out_hbm.at[:, :, pl.ds(base_r, HALF)])
            return carry

        lax.fori_loop(0, 2, do_half, 0)

    return gather_kernel(idx, height_w)


def _height_matmul(latent, g3):
    OB = 8
    NBR = 512

    def body(lat_ref, g_ref, out_ref):
        for oo in range(OB):
            out_ref[:, oo, :] = jnp.dot(
                lat_ref[...], g_ref[:, oo, :], preferred_element_type=jnp.float32
            )

    return pl.pallas_call(
        body,
        grid=(N_OC // OB, R // NBR),
        in_specs=[
            pl.BlockSpec((B, N_LATENT), lambda o, j: (0, 0)),
            pl.BlockSpec((N_LATENT, OB, NBR), lambda o, j: (0, o, j)),
        ],
        out_specs=pl.BlockSpec((B, OB, NBR), lambda o, j: (0, o, j)),
        out_shape=jax.ShapeDtypeStruct((B, N_OC, R), jnp.float32),
    )(latent, g3)


def _baseline_matmul_t(bwT, latent):
    NB = 4096
    n_regions = bwT.shape[1]

    def body(bw_ref, lat_ref, out_ref):
        out_ref[...] = lax.dot_general(
            bw_ref[...],
            lat_ref[...],
            dimension_numbers=(((0,), (1,)), ((), ())),
            preferred_element_type=jnp.float32,
        )

    return pl.pallas_call(
        body,
        grid=(pl.cdiv(n_regions, NB),),
        in_specs=[
            pl.BlockSpec((N_LATENT, NB), lambda i: (0, i)),
            pl.BlockSpec((B, N_LATENT), lambda i: (0, 0)),
        ],
        out_specs=pl.BlockSpec((NB, B), lambda i: (i, 0)),
        out_shape=jax.ShapeDtypeStruct((n_regions, B), jnp.float32),
    )(bwT, latent)


def kernel(latent, regions_oi, height_w, baseline_w):
    g3 = _sc_gather_transposed(regions_oi, height_w)
    dh_p = _height_matmul(latent, g3)
    delta_height = jnp.transpose(dh_p, (0, 2, 1))
    db_t = _baseline_matmul_t(baseline_w.T, latent)
    delta_baseline = db_t.T
    return delta_height, delta_baseline


# speedup vs baseline: 5.7356x; 4.2508x over previous
import functools

import jax
import jax.numpy as jnp
from jax import lax
from jax.experimental import pallas as pl
from jax.experimental.pallas import tpu as pltpu
from jax.experimental.pallas import tpu_sc as plsc

N_LATENT = 32
N_OC = 16
B = 512
R = 4096
LANES = 16
N_WORKERS = 32
RPW = R // N_WORKERS
HALF = RPW // 2


KO = N_LATENT * N_OC


def _sc_gather(idx, hh):
    mesh = plsc.VectorSubcoreMesh(core_axis_name="c", subcore_axis_name="s")

    @functools.partial(
        pl.kernel,
        mesh=mesh,
        out_type=jax.ShapeDtypeStruct((R, KO), jnp.float32),
        scratch_types=[
            pltpu.VMEM((RPW,), jnp.int32),
            pltpu.VMEM((RPW, KO), jnp.float32),
            pltpu.SemaphoreType.DMA,
        ],
        compiler_params=pltpu.CompilerParams(
            use_tc_tiling_on_sc=True, needs_layout_passes=False
        ),
    )
    def gather_kernel(idx_hbm, hh_hbm, out_hbm, idx_v, rows_v, sem):
        w = lax.axis_index("s") * 2 + lax.axis_index("c")
        pltpu.sync_copy(idx_hbm.at[pl.ds(w * RPW, RPW)], idx_v)
        pltpu.async_copy(hh_hbm.at[idx_v], rows_v, sem).wait()
        pltpu.sync_copy(rows_v, out_hbm.at[pl.ds(w * RPW, RPW)])

    return gather_kernel(idx, hh)


def _height_matmul(latent, g3):
    OB = 8
    NBR = 512

    def body(lat_ref, g_ref, out_ref):
        for oo in range(OB):
            out_ref[:, oo, :] = jnp.dot(
                lat_ref[...], g_ref[:, oo, :], preferred_element_type=jnp.float32
            )

    return pl.pallas_call(
        body,
        grid=(N_OC // OB, R // NBR),
        in_specs=[
            pl.BlockSpec((B, N_LATENT), lambda o, j: (0, 0)),
            pl.BlockSpec((N_LATENT, OB, NBR), lambda o, j: (0, o, j)),
        ],
        out_specs=pl.BlockSpec((B, OB, NBR), lambda o, j: (0, o, j)),
        out_shape=jax.ShapeDtypeStruct((B, N_OC, R), jnp.float32),
    )(latent, g3)


def _baseline_matmul_t(bwT, latent):
    NB = 4096
    n_regions = bwT.shape[1]

    def body(bw_ref, lat_ref, out_ref):
        out_ref[...] = lax.dot_general(
            bw_ref[...],
            lat_ref[...],
            dimension_numbers=(((0,), (1,)), ((), ())),
            preferred_element_type=jnp.float32,
        )

    return pl.pallas_call(
        body,
        grid=(pl.cdiv(n_regions, NB),),
        in_specs=[
            pl.BlockSpec((N_LATENT, NB), lambda i: (0, i)),
            pl.BlockSpec((B, N_LATENT), lambda i: (0, 0)),
        ],
        out_specs=pl.BlockSpec((NB, B), lambda i: (i, 0)),
        out_shape=jax.ShapeDtypeStruct((n_regions, B), jnp.float32),
    )(bwT, latent)


def kernel(latent, regions_oi, height_w, baseline_w):
    n_regions = height_w.shape[0]
    hh = height_w.reshape(n_regions, KO)
    g_raw = _sc_gather(regions_oi, hh)
    g3 = jnp.transpose(g_raw.reshape(R, N_LATENT, N_OC), (1, 2, 0))
    dh_p = _height_matmul(latent, g3)
    delta_height = jnp.transpose(dh_p, (0, 2, 1))
    db_t = _baseline_matmul_t(baseline_w.T, latent)
    delta_baseline = db_t.T
    return delta_height, delta_baseline


# speedup vs baseline: 8.4061x; 1.4656x over previous
import functools

import jax
import jax.numpy as jnp
from jax import lax
from jax.experimental import pallas as pl
from jax.experimental.pallas import tpu as pltpu
from jax.experimental.pallas import tpu_sc as plsc

N_LATENT = 32
N_OC = 16
B = 512
R = 4096
LANES = 16
N_WORKERS = 32

KO = N_LATENT * N_OC
KPW = KO // N_WORKERS


def _sc_gather_lanes(idx, wt):
    n_regions = wt.shape[1]
    mesh = plsc.VectorSubcoreMesh(core_axis_name="c", subcore_axis_name="s")

    @functools.partial(
        pl.kernel,
        mesh=mesh,
        out_type=jax.ShapeDtypeStruct((KO, R), jnp.float32),
        scratch_types=[
            pltpu.VMEM((R,), jnp.int32),
            pltpu.VMEM((n_regions,), jnp.float32),
            pltpu.VMEM((R,), jnp.float32),
        ],
        compiler_params=pltpu.CompilerParams(
            use_tc_tiling_on_sc=True, needs_layout_passes=False
        ),
    )
    def gather_kernel(idx_hbm, wt_hbm, out_hbm, idx_v, row_v, orow_v):
        w = lax.axis_index("s") * 2 + lax.axis_index("c")
        pltpu.sync_copy(idx_hbm, idx_v)
        for t in range(KPW):
            k = w * KPW + t
            pltpu.sync_copy(wt_hbm.at[k], row_v)

            def body(j, _):
                iv = idx_v[pl.ds(j * LANES, LANES)]
                orow_v[pl.ds(j * LANES, LANES)] = plsc.load_gather(row_v, [iv])
                return 0

            lax.fori_loop(0, R // LANES, body, 0)
            pltpu.sync_copy(orow_v, out_hbm.at[k])

    return gather_kernel(idx, wt)


def _height_matmul(latent, g3):
    OB = 8
    NBR = 512

    def body(lat_ref, g_ref, out_ref):
        for oo in range(OB):
            out_ref[:, oo, :] = jnp.dot(
                lat_ref[...], g_ref[:, oo, :], preferred_element_type=jnp.float32
            )

    return pl.pallas_call(
        body,
        grid=(N_OC // OB, R // NBR),
        in_specs=[
            pl.BlockSpec((B, N_LATENT), lambda o, j: (0, 0)),
            pl.BlockSpec((N_LATENT, OB, NBR), lambda o, j: (0, o, j)),
        ],
        out_specs=pl.BlockSpec((B, OB, NBR), lambda o, j: (0, o, j)),
        out_shape=jax.ShapeDtypeStruct((B, N_OC, R), jnp.float32),
    )(latent, g3)


def _baseline_matmul_t(bwT, latent):
    NB = 4096
    n_regions = bwT.shape[1]

    def body(bw_ref, lat_ref, out_ref):
        out_ref[...] = lax.dot_general(
            bw_ref[...],
            lat_ref[...],
            dimension_numbers=(((0,), (1,)), ((), ())),
            preferred_element_type=jnp.float32,
        )

    return pl.pallas_call(
        body,
        grid=(pl.cdiv(n_regions, NB),),
        in_specs=[
            pl.BlockSpec((N_LATENT, NB), lambda i: (0, i)),
            pl.BlockSpec((B, N_LATENT), lambda i: (0, 0)),
        ],
        out_specs=pl.BlockSpec((NB, B), lambda i: (i, 0)),
        out_shape=jax.ShapeDtypeStruct((n_regions, B), jnp.float32),
    )(bwT, latent)


def kernel(latent, regions_oi, height_w, baseline_w):
    n_regions = height_w.shape[0]
    wt = jnp.transpose(height_w, (1, 2, 0)).reshape(KO, n_regions)
    g_t = _sc_gather_lanes(regions_oi, wt)
    g3 = g_t.reshape(N_LATENT, N_OC, R)
    dh_p = _height_matmul(latent, g3)
    delta_height = jnp.transpose(dh_p, (0, 2, 1))
    db_t = _baseline_matmul_t(baseline_w.T, latent)
    delta_baseline = db_t.T
    return delta_height, delta_baseline
